# Initial kernel scaffold; baseline (speedup 1.0000x reference)
import jax
import jax.numpy as jnp
from jax.experimental import pallas as pl


def kernel(x, edge_index, batch, conv_params, outer_bn, mlp_params):
    raise NotImplementedError("write your pallas kernel here")


# speedup vs baseline: 5.1978x; 5.1978x over previous
import functools

import jax
import jax.numpy as jnp
from jax import lax
from jax.experimental import pallas as pl
from jax.experimental.pallas import tpu as pltpu
from jax.experimental.pallas import tpu_sc as plsc

_N = 10000
_IN = 128
_H = 64
_OUT = 2
_G = 128
_E = 320000

_NC = 2
_NS = 16
_NW = _NC * _NS

_C = 128
_EPW = -(-_E // (_NW * _C)) * _C
_NCH = _EPW // _C
_EP = _EPW * _NW
_NP = 10240
_RT = _NP // _NS


def _make_seg_sum(F):
  mesh = plsc.VectorSubcoreMesh(core_axis_name="c", subcore_axis_name="s")

  @functools.partial(
      pl.kernel,
      out_type=jax.ShapeDtypeStruct((_NC, _NP, F), jnp.float32),
      mesh=mesh,
      scratch_types=[
          pltpu.VMEM((_NCH, _C), jnp.int32),
          pltpu.VMEM((_NCH, _C), jnp.int32),
          pltpu.VMEM((_C, F), jnp.float32),
          pltpu.VMEM_SHARED((_NP, F), jnp.float32),
          pltpu.SemaphoreType.DMA,
      ],
      compiler_params=pltpu.CompilerParams(use_tc_tiling_on_sc=False),
  )
  def seg_sum(h_hbm, srcb_hbm, dstb_hbm, zb_hbm, out_hbm,
              src_v, dst_v, rows_v, acc_sh, sem):
    c = lax.axis_index("c")
    s = lax.axis_index("s")
    wid = s * _NC + c
    pltpu.sync_copy(srcb_hbm.at[wid], src_v)
    pltpu.sync_copy(dstb_hbm.at[wid], dst_v)
    pltpu.sync_copy(zb_hbm, acc_sh.at[pl.ds(s * _RT, _RT)])
    plsc.subcore_barrier()

    def chunk(j, carry):
      pltpu.async_copy(h_hbm.at[src_v.at[j]], rows_v, sem).wait()
      pltpu.sync_copy(rows_v, acc_sh.at[dst_v.at[j]], add=True)
      return carry

    lax.fori_loop(0, _NCH, chunk, 0)
    plsc.subcore_barrier()
    pltpu.sync_copy(acc_sh.at[pl.ds(s * _RT, _RT)],
                    out_hbm.at[c].at[pl.ds(s * _RT, _RT)])

  return seg_sum


_seg_sum_cache = {}


def _seg_sum(F):
  if F not in _seg_sum_cache:
    _seg_sum_cache[F] = _make_seg_sum(F)
  return _seg_sum_cache[F]


def _bn(h, g, b):
  mu = jnp.mean(h, axis=0, keepdims=True)
  var = jnp.mean((h - mu) ** 2, axis=0, keepdims=True)
  return g * (h - mu) / jnp.sqrt(var + 1e-5) + b


def _tc_layer_body(h_ref, agg_ref, w0, b0, g0, be0, w1, b1, g1, be1,
                   w2, b2, og, obeta, out_ref):
  z = h_ref[...] + agg_ref[0, :_N, :] + agg_ref[1, :_N, :]
  h = jnp.dot(z, w0[...], preferred_element_type=jnp.float32) + b0[...]
  h = jnp.maximum(_bn(h, g0[...], be0[...]), 0.0)
  h = jnp.dot(h, w1[...], preferred_element_type=jnp.float32) + b1[...]
  h = jnp.maximum(_bn(h, g1[...], be1[...]), 0.0)
  h = jnp.dot(h, w2[...], preferred_element_type=jnp.float32) + b2[...]
  h = jnp.maximum(_bn(h, og[...], obeta[...]), 0.0)
  out_ref[...] = h


def _tc_pool_body(*refs):
  x_ref, h1_ref, h2_ref, h3_ref, batch_ref = refs[:5]
  wrefs = refs[5:29]
  out_ref = refs[29]
  b = batch_ref[...]
  gid = lax.broadcasted_iota(jnp.int32, (_G, _N), 0)
  p = (gid == b).astype(jnp.float32)
  hiddens = (x_ref[...], h1_ref[...], h2_ref[...], h3_ref[...])
  score = jnp.zeros((_G, _OUT), jnp.float32)
  for i in range(4):
    pooled = jnp.dot(p, hiddens[i], preferred_element_type=jnp.float32)
    w0, b0, w1, b1, w2, b2 = (wrefs[6 * i + j][...] for j in range(6))
    t = jnp.maximum(jnp.dot(pooled, w0, preferred_element_type=jnp.float32) + b0, 0.0)
    t = jnp.maximum(jnp.dot(t, w1, preferred_element_type=jnp.float32) + b1, 0.0)
    score = score + jnp.dot(t, w2, preferred_element_type=jnp.float32) + b2
  out_ref[...] = score


def kernel(x, edge_index, batch, conv_params, outer_bn, mlp_params):
  src, dst = edge_index[0], edge_index[1]
  pad = _EP - _E
  srcb = jnp.concatenate([src, jnp.zeros((pad,), jnp.int32)]).reshape(_NW, _NCH, _C)
  dstb = jnp.concatenate([dst, jnp.full((pad,), _N, jnp.int32)]).reshape(_NW, _NCH, _C)
  batch2 = batch.reshape(1, _N)

  hidden = [x]
  h = x
  for i in range(3):
    F = _IN if i == 0 else _H
    zb = jnp.zeros((_RT, F), jnp.float32)
    agg = _seg_sum(F)(h, srcb, dstb, zb)
    cp, ob = conv_params[i], outer_bn[i]
    args = (
        h, agg,
        cp['W'][0], cp['b'][0].reshape(1, -1),
        cp['gamma'][0].reshape(1, -1), cp['beta'][0].reshape(1, -1),
        cp['W'][1], cp['b'][1].reshape(1, -1),
        cp['gamma'][1].reshape(1, -1), cp['beta'][1].reshape(1, -1),
        cp['W'][2], cp['b'][2].reshape(1, -1),
        ob['gamma'].reshape(1, -1), ob['beta'].reshape(1, -1),
    )
    h = pl.pallas_call(
        _tc_layer_body,
        out_shape=jax.ShapeDtypeStruct((_N, _H), jnp.float32),
    )(*args)
    hidden.append(h)

  wargs = []
  for i in range(4):
    mp = mlp_params[i]
    for j in range(3):
      wargs += [mp['W'][j], mp['b'][j].reshape(1, -1)]
  score = pl.pallas_call(
      _tc_pool_body,
      out_shape=jax.ShapeDtypeStruct((_G, _OUT), jnp.float32),
  )(hidden[0], hidden[1], hidden[2], hidden[3], batch2, *wargs)
  return score
